# speedup vs baseline: 19.6161x; 19.6161x over previous
import jax
import jax.numpy as jnp
from jax.experimental import pallas as pl
from jax.experimental.pallas import tpu as pltpu

E = 8
K = 2
H = 1024
I = 2048
T = 2048
EPAD = 128

BI = 512


def _router_body(x_ref, rw_ref, gates_ref):
    x = x_ref[...]
    rw = rw_ref[...]
    logits = jax.lax.dot_general(
        x, rw, (((1,), (1,)), ((), ())), preferred_element_type=jnp.float32
    )
    lane = jax.lax.broadcasted_iota(jnp.int32, (T, EPAD), 1)
    valid = lane < E
    logits = jnp.where(valid, logits, -1e30)
    m = jnp.max(logits, axis=1, keepdims=True)
    p = jnp.exp(logits - m)
    p = jnp.where(valid, p, 0.0)
    s = jnp.sum(p, axis=1, keepdims=True)
    probs = p / s
    m1 = jnp.max(probs, axis=1, keepdims=True)
    idx1 = jnp.min(jnp.where(probs == m1, lane, EPAD), axis=1, keepdims=True)
    probs2 = jnp.where(lane == idx1, -1.0, probs)
    m2 = jnp.max(probs2, axis=1, keepdims=True)
    idx2 = jnp.min(jnp.where(probs2 == m2, lane, EPAD), axis=1, keepdims=True)
    keep = (lane == idx1) | (lane == idx2)
    gates_ref[...] = jnp.where(keep, probs, 0.0)


def _expert_body(x_ref, gates_ref, w1_ref, w2_ref, out_ref):
    e = pl.program_id(0)
    c = pl.program_id(1)

    @pl.when((e == 0) & (c == 0))
    def _():
        out_ref[...] = jnp.zeros_like(out_ref)

    x = x_ref[...].astype(jnp.bfloat16)
    wblk = w1_ref[0]
    wg = wblk[:, 0, :].astype(jnp.bfloat16)
    wu = wblk[:, 1, :].astype(jnp.bfloat16)
    g = jax.lax.dot_general(
        x, wg, (((1,), (1,)), ((), ())), preferred_element_type=jnp.float32
    )
    u = jax.lax.dot_general(
        x, wu, (((1,), (1,)), ((), ())), preferred_element_type=jnp.float32
    )
    a = (u * (g * jax.nn.sigmoid(g))).astype(jnp.bfloat16)
    w2b = w2_ref[0].astype(jnp.bfloat16)
    partial = jax.lax.dot_general(
        a, w2b, (((1,), (1,)), ((), ())), preferred_element_type=jnp.float32
    )
    lane = jax.lax.broadcasted_iota(jnp.int32, (T, EPAD), 1)
    gcol = jnp.sum(
        jnp.where(lane == e, gates_ref[...], 0.0), axis=1, keepdims=True
    )
    out_ref[...] += gcol * partial


def kernel(hidden_states, router_w, w1, w2):
    orig_shape = hidden_states.shape
    xf = hidden_states.reshape(-1, orig_shape[-1])
    rw_pad = jnp.zeros((EPAD, H), jnp.float32).at[:E].set(router_w)

    gates = pl.pallas_call(
        _router_body,
        out_shape=jax.ShapeDtypeStruct((T, EPAD), jnp.float32),
        in_specs=[
            pl.BlockSpec((T, H), lambda: (0, 0)),
            pl.BlockSpec((EPAD, H), lambda: (0, 0)),
        ],
        out_specs=pl.BlockSpec((T, EPAD), lambda: (0, 0)),
    )(xf, rw_pad)

    w1r = w1.reshape(E, I, 2, H)
    out = pl.pallas_call(
        _expert_body,
        grid=(E, I // BI),
        out_shape=jax.ShapeDtypeStruct((T, H), jnp.float32),
        in_specs=[
            pl.BlockSpec((T, H), lambda e, c: (0, 0)),
            pl.BlockSpec((T, EPAD), lambda e, c: (0, 0)),
            pl.BlockSpec((1, BI, 2, H), lambda e, c: (e, c, 0, 0)),
            pl.BlockSpec((1, H, BI), lambda e, c: (e, 0, c)),
        ],
        out_specs=pl.BlockSpec((T, H), lambda e, c: (0, 0)),
        compiler_params=pltpu.CompilerParams(
            dimension_semantics=("arbitrary", "arbitrary"),
        ),
    )(xf, gates, w1r, w2)
    return out.reshape(orig_shape)
